# baseline (speedup 1.0000x reference)
import jax
import jax.numpy as jnp
import numpy as np
from jax.experimental import pallas as pl

_W = 512
_TWO_W = 1024
_OMEGA = 100
_N_ITERS = 8
_BATCH = 256
_NPV = 16
_SQRT_W = float(np.sqrt(_W))


def _hard_thr(c):
    a = jax.lax.bitcast_convert_type(jnp.abs(c), jnp.int32)
    rows = c.shape[0]

    def count(cand):
        m = (a >= cand).astype(jnp.int32)
        s = m[:, 0:128]
        for k in range(1, 8):
            s = s + m[:, 128 * k:128 * (k + 1)]
        return jnp.sum(s, axis=1, keepdims=True)

    def count3(c1, c2, c3):
        s1 = s2 = s3 = None
        for k in range(8):
            ak = a[:, 128 * k:128 * (k + 1)]
            m1 = (ak >= c1).astype(jnp.int32)
            m2 = (ak >= c2).astype(jnp.int32)
            m3 = (ak >= c3).astype(jnp.int32)
            s1 = m1 if s1 is None else s1 + m1
            s2 = m2 if s2 is None else s2 + m2
            s3 = m3 if s3 is None else s3 + m3
        return (jnp.sum(s1, axis=1, keepdims=True),
                jnp.sum(s2, axis=1, keepdims=True),
                jnp.sum(s3, axis=1, keepdims=True))

    top = jnp.int32(1 << 30)
    cnt30 = count(top)
    take30 = cnt30 >= _OMEGA
    t0 = jnp.where(take30, top, 0)
    cnt_t0 = jnp.where(take30, cnt30, jnp.full((rows, 1), a.shape[1], jnp.int32))

    def cond(state):
        bit, _, cnt_t = state
        return jnp.logical_and(bit >= 0, jnp.any(cnt_t > _OMEGA))

    def body(state):
        bit, t, cnt_t = state
        b_hi = jnp.left_shift(jnp.int32(1), bit)
        b_lo = jnp.left_shift(jnp.int32(1), bit - 1)
        c1 = jnp.bitwise_or(t, b_hi)
        c12 = jnp.bitwise_or(c1, b_lo)
        c2 = jnp.bitwise_or(t, b_lo)
        n1, n12, n2 = count3(c1, c12, c2)
        take1 = n1 >= _OMEGA
        cnt_lo = jnp.where(take1, n12, n2)
        take2 = cnt_lo >= _OMEGA
        t_new = jnp.where(take2,
                          jnp.where(take1, c12, c2),
                          jnp.where(take1, c1, t))
        cnt_new = jnp.where(take2, cnt_lo, jnp.where(take1, n1, cnt_t))
        return bit - 2, t_new, cnt_new

    _, t, _ = jax.lax.while_loop(cond, body, (jnp.int32(29), t0, cnt_t0))
    return jnp.where(a >= t, c, 0.0)


def _star_kernel(x_ref, pw_ref, wd_ref, wm_ref, wa_ref, ba_ref, md_ref, z_ref):
    f32 = jnp.float32
    wd = wd_ref[...]
    x = x_ref[...]

    b_mat = jax.lax.dot_general(x, 0.5 * wd, (((1,), (1,)), ((), ())),
                                preferred_element_type=f32)
    g = jax.lax.dot_general(wd, wd, (((0,), (0,)), ((), ())),
                            preferred_element_type=f32)
    ii = jax.lax.broadcasted_iota(jnp.int32, (_TWO_W, _TWO_W), 0)
    jj = jax.lax.broadcasted_iota(jnp.int32, (_TWO_W, _TWO_W), 1)
    s_mat = jnp.where(ii == jj, f32(1.0), f32(0.0)) - 0.5 * g

    z = _hard_thr(b_mat)

    def iter_body(_, z):
        c = b_mat + jax.lax.dot_general(z, s_mat, (((1,), (1,)), ((), ())),
                                        preferred_element_type=f32)
        return _hard_thr(c)

    z = jax.lax.fori_loop(0, _N_ITERS, iter_body, z)
    z_ref[...] = z

    p = z[:, :_W] * z[:, :_W] + z[:, _W:] * z[:, _W:]
    md = jnp.sum(p, axis=0, keepdims=True)
    md_n = (md - jnp.min(md)) / (jnp.max(md) - jnp.min(md) + 1e-8)

    rows = []
    for n in range(_NPV):
        h = pw_ref[n * _BATCH:(n + 1) * _BATCH, :]
        ph = h[:, :_W] * h[:, :_W] + h[:, _W:] * h[:, _W:]
        rows.append(jnp.sum(ph, axis=0, keepdims=True))
    mdp = jnp.concatenate(rows, axis=0)
    lo = jnp.min(mdp, axis=1, keepdims=True)
    hi = jnp.max(mdp, axis=1, keepdims=True)
    mdp_n = (mdp - lo) / (hi - lo + 1e-8)

    att = jnp.sum(mdp_n * md_n, axis=1, keepdims=True)
    att = att / _SQRT_W
    e = jnp.exp(att - jnp.max(att))
    sm = e / jnp.sum(e)
    a = jnp.sum(mdp_n * sm, axis=0, keepdims=True)

    am = jax.nn.sigmoid(jax.lax.dot_general(a, wm_ref[...], (((1,), (1,)), ((), ())),
                                            preferred_element_type=f32))
    aa = jax.nn.relu(jax.lax.dot_general(a, wa_ref[...], (((1,), (1,)), ((), ())),
                                         preferred_element_type=f32) + ba_ref[...])
    mo = (md_n + aa) * am
    md_ref[...] = (mo - jnp.min(mo)) / (jnp.max(mo) - jnp.min(mo) + 1e-8)


def _star_call(x, pw, wd, wm, wa, ba, interpret=False):
    return pl.pallas_call(
        _star_kernel,
        out_shape=(
            jax.ShapeDtypeStruct((1, _W), jnp.float32),
            jax.ShapeDtypeStruct((_BATCH, _TWO_W), jnp.float32),
        ),
        interpret=interpret,
    )(x, pw, wd, wm, wa, ba)


def kernel(x, prev_windows, W_d, Wm, Wa, ba):
    pw = prev_windows.reshape(-1, prev_windows.shape[-1])
    md, z = _star_call(x, pw, W_d[0], Wm, Wa, ba.reshape(1, -1))
    return md.reshape(-1), z


# speedup vs baseline: 1.2131x; 1.2131x over previous
import jax
import jax.numpy as jnp
import numpy as np
from jax.experimental import pallas as pl

_W = 512
_TWO_W = 1024
_OMEGA = 100
_N_ITERS = 8
_BATCH = 256
_NPV = 16
_SQRT_W = float(np.sqrt(_W))


def _hard_thr_t(ct):
    a = jax.lax.bitcast_convert_type(jnp.abs(ct), jnp.int32)
    n = a.shape[0]
    cols = a.shape[1]

    def count1(cand):
        m = (a >= cand).astype(jnp.int32)
        s = m[0:128, :]
        for k in range(1, 8):
            s = s + m[128 * k:128 * (k + 1), :]
        return jnp.sum(s, axis=0, keepdims=True)

    def count3(c1, c2, c3):
        s1 = s2 = s3 = None
        for k in range(8):
            ak = a[128 * k:128 * (k + 1), :]
            m1 = (ak >= c1).astype(jnp.int32)
            m2 = (ak >= c2).astype(jnp.int32)
            m3 = (ak >= c3).astype(jnp.int32)
            s1 = m1 if s1 is None else s1 + m1
            s2 = m2 if s2 is None else s2 + m2
            s3 = m3 if s3 is None else s3 + m3
        return (jnp.sum(s1, axis=0, keepdims=True),
                jnp.sum(s2, axis=0, keepdims=True),
                jnp.sum(s3, axis=0, keepdims=True))

    top = jnp.int32(1 << 30)
    cnt30 = count1(top)
    take30 = cnt30 >= _OMEGA
    t0 = jnp.where(take30, top, 0)
    cnt_t0 = jnp.where(take30, cnt30, jnp.full((1, cols), n, jnp.int32))

    def cond(state):
        bit, _, cnt_t = state
        return jnp.logical_and(bit >= 0, jnp.any(cnt_t > _OMEGA))

    def body(state):
        bit, t, cnt_t = state
        b_hi = jnp.left_shift(jnp.int32(1), bit)
        b_lo = jnp.left_shift(jnp.int32(1), bit - 1)
        c1 = jnp.bitwise_or(t, b_hi)
        c12 = jnp.bitwise_or(c1, b_lo)
        c2 = jnp.bitwise_or(t, b_lo)
        n1, n12, n2 = count3(c1, c12, c2)
        take1 = n1 >= _OMEGA
        cnt_lo = jnp.where(take1, n12, n2)
        take2 = cnt_lo >= _OMEGA
        t_new = jnp.where(take2,
                          jnp.where(take1, c12, c2),
                          jnp.where(take1, c1, t))
        cnt_new = jnp.where(take2, cnt_lo, jnp.where(take1, n1, cnt_t))
        return bit - 2, t_new, cnt_new

    _, t, _ = jax.lax.while_loop(cond, body, (jnp.int32(29), t0, cnt_t0))
    return jnp.where(a >= t, ct, 0.0)


def _star_kernel(x_ref, pw_ref, wd_ref, wm_ref, wa_ref, ba_ref, md_ref, z_ref):
    f32 = jnp.float32
    wd = wd_ref[...]
    x = x_ref[...]

    bt = jax.lax.dot_general(0.5 * wd, x, (((1,), (1,)), ((), ())),
                             preferred_element_type=f32)
    g = jax.lax.dot_general(wd, wd, (((0,), (0,)), ((), ())),
                            preferred_element_type=f32)
    ii = jax.lax.broadcasted_iota(jnp.int32, (_TWO_W, _TWO_W), 0)
    jj = jax.lax.broadcasted_iota(jnp.int32, (_TWO_W, _TWO_W), 1)
    s_mat = jnp.where(ii == jj, f32(1.0), f32(0.0)) - 0.5 * g

    zt = _hard_thr_t(bt)

    def iter_body(_, zt):
        ct = bt + jax.lax.dot_general(s_mat, zt, (((1,), (0,)), ((), ())),
                                      preferred_element_type=f32)
        return _hard_thr_t(ct)

    zt = jax.lax.fori_loop(0, _N_ITERS, iter_body, zt)
    z = jnp.swapaxes(zt, 0, 1)
    z_ref[...] = z

    p = z[:, :_W] * z[:, :_W] + z[:, _W:] * z[:, _W:]
    md = jnp.sum(p, axis=0, keepdims=True)
    md_n = (md - jnp.min(md)) / (jnp.max(md) - jnp.min(md) + 1e-8)

    rows = []
    for w in range(_NPV):
        h = pw_ref[w * _BATCH:(w + 1) * _BATCH, :]
        ph = h[:, :_W] * h[:, :_W] + h[:, _W:] * h[:, _W:]
        rows.append(jnp.sum(ph, axis=0, keepdims=True))
    mdp = jnp.concatenate(rows, axis=0)
    lo = jnp.min(mdp, axis=1, keepdims=True)
    hi = jnp.max(mdp, axis=1, keepdims=True)
    mdp_n = (mdp - lo) / (hi - lo + 1e-8)

    att = jnp.sum(mdp_n * md_n, axis=1, keepdims=True)
    att = att / _SQRT_W
    e = jnp.exp(att - jnp.max(att))
    sm = e / jnp.sum(e)
    a = jnp.sum(mdp_n * sm, axis=0, keepdims=True)

    am = jax.nn.sigmoid(jax.lax.dot_general(a, wm_ref[...], (((1,), (1,)), ((), ())),
                                            preferred_element_type=f32))
    aa = jax.nn.relu(jax.lax.dot_general(a, wa_ref[...], (((1,), (1,)), ((), ())),
                                         preferred_element_type=f32) + ba_ref[...])
    mo = (md_n + aa) * am
    md_ref[...] = (mo - jnp.min(mo)) / (jnp.max(mo) - jnp.min(mo) + 1e-8)


def _star_call(x, pw, wd, wm, wa, ba, interpret=False):
    return pl.pallas_call(
        _star_kernel,
        out_shape=(
            jax.ShapeDtypeStruct((1, _W), jnp.float32),
            jax.ShapeDtypeStruct((_BATCH, _TWO_W), jnp.float32),
        ),
        interpret=interpret,
    )(x, pw, wd, wm, wa, ba)


def kernel(x, prev_windows, W_d, Wm, Wa, ba):
    pw = prev_windows.reshape(-1, prev_windows.shape[-1])
    md, z = _star_call(x, pw, W_d[0], Wm, Wa, ba.reshape(1, -1))
    return md.reshape(-1), z


# speedup vs baseline: 1.2607x; 1.0392x over previous
import jax
import jax.numpy as jnp
import numpy as np
from jax.experimental import pallas as pl

_W = 512
_TWO_W = 1024
_OMEGA = 100
_N_ITERS = 8
_BATCH = 256
_NPV = 16
_SQRT_W = float(np.sqrt(_W))


def _hard_thr_t(ct):
    a = jax.lax.bitcast_convert_type(jnp.abs(ct), jnp.int32)
    n = a.shape[0]
    cols = a.shape[1]

    def count1(cand):
        m = (a >= cand).astype(jnp.int32)
        s = m[0:128, :]
        for k in range(1, 8):
            s = s + m[128 * k:128 * (k + 1), :]
        return jnp.sum(s, axis=0, keepdims=True)

    def count3(c1, c2, c3):
        s1 = s2 = s3 = None
        for k in range(8):
            ak = a[128 * k:128 * (k + 1), :]
            m1 = (ak >= c1).astype(jnp.int32)
            m2 = (ak >= c2).astype(jnp.int32)
            m3 = (ak >= c3).astype(jnp.int32)
            s1 = m1 if s1 is None else s1 + m1
            s2 = m2 if s2 is None else s2 + m2
            s3 = m3 if s3 is None else s3 + m3
        return (jnp.sum(s1, axis=0, keepdims=True),
                jnp.sum(s2, axis=0, keepdims=True),
                jnp.sum(s3, axis=0, keepdims=True))

    top = jnp.int32(1 << 30)
    cnt30 = count1(top)
    take30 = cnt30 >= _OMEGA
    t0 = jnp.where(take30, top, 0)
    cnt_t0 = jnp.where(take30, cnt30, jnp.full((1, cols), n, jnp.int32))

    def cond(state):
        bit, _, cnt_t = state
        return jnp.logical_and(bit >= 3, jnp.any(cnt_t > _OMEGA))

    def pair_step(bit, t, cnt_t):
        b_hi = jnp.left_shift(jnp.int32(1), bit)
        b_lo = jnp.left_shift(jnp.int32(1), bit - 1)
        c1 = jnp.bitwise_or(t, b_hi)
        c12 = jnp.bitwise_or(c1, b_lo)
        c2 = jnp.bitwise_or(t, b_lo)
        n1, n12, n2 = count3(c1, c12, c2)
        take1 = n1 >= _OMEGA
        cnt_lo = jnp.where(take1, n12, n2)
        take2 = cnt_lo >= _OMEGA
        t_new = jnp.where(take2,
                          jnp.where(take1, c12, c2),
                          jnp.where(take1, c1, t))
        cnt_new = jnp.where(take2, cnt_lo, jnp.where(take1, n1, cnt_t))
        return t_new, cnt_new

    def body(state):
        bit, t, cnt_t = state
        t, cnt_t = pair_step(bit, t, cnt_t)
        t, cnt_t = pair_step(bit - 2, t, cnt_t)
        return bit - 4, t, cnt_t

    _, t, cnt_t = jax.lax.while_loop(cond, body, (jnp.int32(29), t0, cnt_t0))
    t = jax.lax.cond(jnp.any(cnt_t > _OMEGA),
                     lambda tc: pair_step(jnp.int32(1), tc[0], tc[1])[0],
                     lambda tc: tc[0],
                     (t, cnt_t))
    return jnp.where(a >= t, ct, 0.0)


def _star_kernel(x_ref, pw_ref, wd_ref, wm_ref, wa_ref, ba_ref, md_ref, z_ref):
    f32 = jnp.float32
    wd = wd_ref[...]
    x = x_ref[...]

    bt = jax.lax.dot_general(0.5 * wd, x, (((1,), (1,)), ((), ())),
                             preferred_element_type=f32)
    g = jax.lax.dot_general(wd, wd, (((0,), (0,)), ((), ())),
                            preferred_element_type=f32)
    ii = jax.lax.broadcasted_iota(jnp.int32, (_TWO_W, _TWO_W), 0)
    jj = jax.lax.broadcasted_iota(jnp.int32, (_TWO_W, _TWO_W), 1)
    s_mat = jnp.where(ii == jj, f32(1.0), f32(0.0)) - 0.5 * g

    zt = _hard_thr_t(bt)

    def iter_body(_, zt):
        ct = bt + jax.lax.dot_general(s_mat, zt, (((1,), (0,)), ((), ())),
                                      preferred_element_type=f32)
        return _hard_thr_t(ct)

    zt = jax.lax.fori_loop(0, _N_ITERS, iter_body, zt)
    z = jnp.swapaxes(zt, 0, 1)
    z_ref[...] = z

    p = z[:, :_W] * z[:, :_W] + z[:, _W:] * z[:, _W:]
    md = jnp.sum(p, axis=0, keepdims=True)
    md_n = (md - jnp.min(md)) / (jnp.max(md) - jnp.min(md) + 1e-8)

    rows = []
    for w in range(_NPV):
        h = pw_ref[w * _BATCH:(w + 1) * _BATCH, :]
        ph = h[:, :_W] * h[:, :_W] + h[:, _W:] * h[:, _W:]
        rows.append(jnp.sum(ph, axis=0, keepdims=True))
    mdp = jnp.concatenate(rows, axis=0)
    lo = jnp.min(mdp, axis=1, keepdims=True)
    hi = jnp.max(mdp, axis=1, keepdims=True)
    mdp_n = (mdp - lo) / (hi - lo + 1e-8)

    att = jnp.sum(mdp_n * md_n, axis=1, keepdims=True)
    att = att / _SQRT_W
    e = jnp.exp(att - jnp.max(att))
    sm = e / jnp.sum(e)
    a = jnp.sum(mdp_n * sm, axis=0, keepdims=True)

    am = jax.nn.sigmoid(jax.lax.dot_general(a, wm_ref[...], (((1,), (1,)), ((), ())),
                                            preferred_element_type=f32))
    aa = jax.nn.relu(jax.lax.dot_general(a, wa_ref[...], (((1,), (1,)), ((), ())),
                                         preferred_element_type=f32) + ba_ref[...])
    mo = (md_n + aa) * am
    md_ref[...] = (mo - jnp.min(mo)) / (jnp.max(mo) - jnp.min(mo) + 1e-8)


def _star_call(x, pw, wd, wm, wa, ba, interpret=False):
    return pl.pallas_call(
        _star_kernel,
        out_shape=(
            jax.ShapeDtypeStruct((1, _W), jnp.float32),
            jax.ShapeDtypeStruct((_BATCH, _TWO_W), jnp.float32),
        ),
        interpret=interpret,
    )(x, pw, wd, wm, wa, ba)


def kernel(x, prev_windows, W_d, Wm, Wa, ba):
    pw = prev_windows.reshape(-1, prev_windows.shape[-1])
    md, z = _star_call(x, pw, W_d[0], Wm, Wa, ba.reshape(1, -1))
    return md.reshape(-1), z


# speedup vs baseline: 1.3614x; 1.0798x over previous
import jax
import jax.numpy as jnp
import numpy as np
from jax.experimental import pallas as pl
from jax.experimental.pallas import tpu as pltpu

_W = 512
_TWO_W = 1024
_OMEGA = 100
_N_ITERS = 8
_BATCH = 256
_NPV = 16
_SQRT_W = float(np.sqrt(_W))


def _hard_thr_t(ct):
    a = jax.lax.bitcast_convert_type(jnp.abs(ct), jnp.int32)
    n = a.shape[0]
    cols = a.shape[1]

    def count1(cand):
        m = (a >= cand).astype(jnp.int32)
        s = m[0:128, :]
        for k in range(1, 8):
            s = s + m[128 * k:128 * (k + 1), :]
        return jnp.sum(s, axis=0, keepdims=True)

    def count3(c1, c2, c3):
        s1 = s2 = s3 = None
        for k in range(8):
            ak = a[128 * k:128 * (k + 1), :]
            m1 = (ak >= c1).astype(jnp.int32)
            m2 = (ak >= c2).astype(jnp.int32)
            m3 = (ak >= c3).astype(jnp.int32)
            s1 = m1 if s1 is None else s1 + m1
            s2 = m2 if s2 is None else s2 + m2
            s3 = m3 if s3 is None else s3 + m3
        return (jnp.sum(s1, axis=0, keepdims=True),
                jnp.sum(s2, axis=0, keepdims=True),
                jnp.sum(s3, axis=0, keepdims=True))

    top = jnp.int32(1 << 30)
    cnt30 = count1(top)
    take30 = cnt30 >= _OMEGA
    t0 = jnp.where(take30, top, 0)
    cnt_t0 = jnp.where(take30, cnt30, jnp.full((1, cols), n, jnp.int32))

    def cond(state):
        bit, _, cnt_t = state
        return jnp.logical_and(bit >= 3, jnp.any(cnt_t > _OMEGA))

    def pair_step(bit, t, cnt_t):
        b_hi = jnp.left_shift(jnp.int32(1), bit)
        b_lo = jnp.left_shift(jnp.int32(1), bit - 1)
        c1 = jnp.bitwise_or(t, b_hi)
        c12 = jnp.bitwise_or(c1, b_lo)
        c2 = jnp.bitwise_or(t, b_lo)
        n1, n12, n2 = count3(c1, c12, c2)
        take1 = n1 >= _OMEGA
        cnt_lo = jnp.where(take1, n12, n2)
        take2 = cnt_lo >= _OMEGA
        t_new = jnp.where(take2,
                          jnp.where(take1, c12, c2),
                          jnp.where(take1, c1, t))
        cnt_new = jnp.where(take2, cnt_lo, jnp.where(take1, n1, cnt_t))
        return t_new, cnt_new

    def body(state):
        bit, t, cnt_t = state
        t, cnt_t = pair_step(bit, t, cnt_t)
        t, cnt_t = pair_step(bit - 2, t, cnt_t)
        return bit - 4, t, cnt_t

    _, t, cnt_t = jax.lax.while_loop(cond, body, (jnp.int32(29), t0, cnt_t0))
    t = jax.lax.cond(jnp.any(cnt_t > _OMEGA),
                     lambda tc: pair_step(jnp.int32(1), tc[0], tc[1])[0],
                     lambda tc: tc[0],
                     (t, cnt_t))
    return jnp.where(a >= t, ct, 0.0)


def _star_kernel(x_ref, pw_ref, wd_ref, wm_ref, wa_ref, ba_ref, md_ref, z_ref,
                 pw_vmem, pw_sem):
    f32 = jnp.float32
    pw_copy = pltpu.make_async_copy(pw_ref, pw_vmem, pw_sem)
    pw_copy.start()
    wd = wd_ref[...]
    x = x_ref[...]

    bt = jax.lax.dot_general(0.5 * wd, x, (((1,), (1,)), ((), ())),
                             preferred_element_type=f32)
    g = jax.lax.dot_general(wd, wd, (((0,), (0,)), ((), ())),
                            preferred_element_type=f32)
    ii = jax.lax.broadcasted_iota(jnp.int32, (_TWO_W, _TWO_W), 0)
    jj = jax.lax.broadcasted_iota(jnp.int32, (_TWO_W, _TWO_W), 1)
    s_mat = jnp.where(ii == jj, f32(1.0), f32(0.0)) - 0.5 * g

    zt = _hard_thr_t(bt)

    def iter_body(_, zt):
        ct = bt + jax.lax.dot_general(s_mat, zt, (((1,), (0,)), ((), ())),
                                      preferred_element_type=f32)
        return _hard_thr_t(ct)

    zt = jax.lax.fori_loop(0, _N_ITERS, iter_body, zt)
    z = jnp.swapaxes(zt, 0, 1)
    z_ref[...] = z

    p = z[:, :_W] * z[:, :_W] + z[:, _W:] * z[:, _W:]
    md = jnp.sum(p, axis=0, keepdims=True)
    md_n = (md - jnp.min(md)) / (jnp.max(md) - jnp.min(md) + 1e-8)

    pw_copy.wait()
    rows = []
    for w in range(_NPV):
        h = pw_vmem[w * _BATCH:(w + 1) * _BATCH, :]
        ph = h[:, :_W] * h[:, :_W] + h[:, _W:] * h[:, _W:]
        rows.append(jnp.sum(ph, axis=0, keepdims=True))
    mdp = jnp.concatenate(rows, axis=0)
    lo = jnp.min(mdp, axis=1, keepdims=True)
    hi = jnp.max(mdp, axis=1, keepdims=True)
    mdp_n = (mdp - lo) / (hi - lo + 1e-8)

    att = jnp.sum(mdp_n * md_n, axis=1, keepdims=True)
    att = att / _SQRT_W
    e = jnp.exp(att - jnp.max(att))
    sm = e / jnp.sum(e)
    a = jnp.sum(mdp_n * sm, axis=0, keepdims=True)

    am = jax.nn.sigmoid(jax.lax.dot_general(a, wm_ref[...], (((1,), (1,)), ((), ())),
                                            preferred_element_type=f32))
    aa = jax.nn.relu(jax.lax.dot_general(a, wa_ref[...], (((1,), (1,)), ((), ())),
                                         preferred_element_type=f32) + ba_ref[...])
    mo = (md_n + aa) * am
    md_ref[...] = (mo - jnp.min(mo)) / (jnp.max(mo) - jnp.min(mo) + 1e-8)


def _star_call(x, pw, wd, wm, wa, ba, interpret=False):
    return pl.pallas_call(
        _star_kernel,
        in_specs=[
            pl.BlockSpec(memory_space=pltpu.MemorySpace.VMEM),
            pl.BlockSpec(memory_space=pltpu.MemorySpace.HBM),
            pl.BlockSpec(memory_space=pltpu.MemorySpace.VMEM),
            pl.BlockSpec(memory_space=pltpu.MemorySpace.VMEM),
            pl.BlockSpec(memory_space=pltpu.MemorySpace.VMEM),
            pl.BlockSpec(memory_space=pltpu.MemorySpace.VMEM),
        ],
        scratch_shapes=[
            pltpu.VMEM((_NPV * _BATCH, _TWO_W), jnp.float32),
            pltpu.SemaphoreType.DMA,
        ],
        out_shape=(
            jax.ShapeDtypeStruct((1, _W), jnp.float32),
            jax.ShapeDtypeStruct((_BATCH, _TWO_W), jnp.float32),
        ),
        interpret=interpret,
    )(x, pw, wd, wm, wa, ba)


def kernel(x, prev_windows, W_d, Wm, Wa, ba):
    pw = prev_windows.reshape(-1, prev_windows.shape[-1])
    md, z = _star_call(x, pw, W_d[0], Wm, Wa, ba.reshape(1, -1))
    return md.reshape(-1), z


# speedup vs baseline: 1.5756x; 1.1573x over previous
import jax
import jax.numpy as jnp
import numpy as np
from jax.experimental import pallas as pl
from jax.experimental.pallas import tpu as pltpu

_W = 512
_TWO_W = 1024
_OMEGA = 100
_N_ITERS = 8
_BATCH = 256
_NPV = 16
_SQRT_W = float(np.sqrt(_W))


def _hard_thr_t(ct):
    a = jax.lax.bitcast_convert_type(jnp.abs(ct), jnp.int32)
    n = a.shape[0]
    cols = a.shape[1]

    def count1(cand):
        m = (a >= cand).astype(jnp.int32)
        s = m[0:128, :]
        for k in range(1, 8):
            s = s + m[128 * k:128 * (k + 1), :]
        return jnp.sum(s, axis=0, keepdims=True)

    def count3(c1, c2, c3):
        s1 = s2 = s3 = None
        for k in range(8):
            ak = a[128 * k:128 * (k + 1), :]
            m1 = (ak >= c1).astype(jnp.int32)
            m2 = (ak >= c2).astype(jnp.int32)
            m3 = (ak >= c3).astype(jnp.int32)
            s1 = m1 if s1 is None else s1 + m1
            s2 = m2 if s2 is None else s2 + m2
            s3 = m3 if s3 is None else s3 + m3
        return (jnp.sum(s1, axis=0, keepdims=True),
                jnp.sum(s2, axis=0, keepdims=True),
                jnp.sum(s3, axis=0, keepdims=True))

    ah = jax.lax.shift_right_logical(a, 16).astype(jnp.int16)
    i16 = jnp.int16

    def count3h(c1, c2, c3):
        c1h, c2h, c3h = c1.astype(i16), c2.astype(i16), c3.astype(i16)
        s1 = s2 = s3 = None
        for k in range(8):
            ak = ah[128 * k:128 * (k + 1), :]
            m1 = jnp.where(ak >= c1h, i16(1), i16(0))
            m2 = jnp.where(ak >= c2h, i16(1), i16(0))
            m3 = jnp.where(ak >= c3h, i16(1), i16(0))
            s1 = m1 if s1 is None else s1 + m1
            s2 = m2 if s2 is None else s2 + m2
            s3 = m3 if s3 is None else s3 + m3
        return (jnp.sum(s1.astype(jnp.int32), axis=0, keepdims=True),
                jnp.sum(s2.astype(jnp.int32), axis=0, keepdims=True),
                jnp.sum(s3.astype(jnp.int32), axis=0, keepdims=True))

    def pair_step_h(bit, t, cnt_t):
        b_hi = jnp.left_shift(jnp.int32(1), bit)
        b_lo = jnp.left_shift(jnp.int32(1), bit - 1)
        c1 = jnp.bitwise_or(t, b_hi)
        c12 = jnp.bitwise_or(c1, b_lo)
        c2 = jnp.bitwise_or(t, b_lo)
        n1, n12, n2 = count3h(c1, c12, c2)
        take1 = n1 >= _OMEGA
        cnt_lo = jnp.where(take1, n12, n2)
        take2 = cnt_lo >= _OMEGA
        t_new = jnp.where(take2,
                          jnp.where(take1, c12, c2),
                          jnp.where(take1, c1, t))
        cnt_new = jnp.where(take2, cnt_lo, jnp.where(take1, n1, cnt_t))
        return t_new, cnt_new

    def count1h(cand):
        ch = cand.astype(i16)
        s = None
        for k in range(8):
            m = jnp.where(ah[128 * k:128 * (k + 1), :] >= ch, i16(1), i16(0))
            s = m if s is None else s + m
        return jnp.sum(s.astype(jnp.int32), axis=0, keepdims=True)

    top = jnp.full((1, cols), 1 << 14, jnp.int32)
    cnt14 = count1h(top)
    take14 = cnt14 >= _OMEGA
    th = jnp.where(take14, top, 0)
    cnt_h = jnp.where(take14, cnt14, jnp.full((1, cols), n, jnp.int32))
    th, cnt_h = pair_step_h(jnp.int32(13), th, cnt_h)

    def cond_h(state):
        bit, _, cnt_t = state
        return jnp.logical_and(bit >= 1, jnp.any(cnt_t > _OMEGA))

    def body_h(state):
        bit, t, cnt_t = state
        t, cnt_t = pair_step_h(bit, t, cnt_t)
        t, cnt_t = pair_step_h(bit - 2, t, cnt_t)
        return bit - 4, t, cnt_t

    _, th, cnt_h = jax.lax.while_loop(cond_h, body_h, (jnp.int32(11), th, cnt_h))

    t0 = jnp.left_shift(th, 16)
    cnt_t0 = cnt_h

    def cond(state):
        bit, _, cnt_t = state
        return jnp.logical_and(bit >= 1, jnp.any(cnt_t > _OMEGA))

    def pair_step(bit, t, cnt_t):
        b_hi = jnp.left_shift(jnp.int32(1), bit)
        b_lo = jnp.left_shift(jnp.int32(1), bit - 1)
        c1 = jnp.bitwise_or(t, b_hi)
        c12 = jnp.bitwise_or(c1, b_lo)
        c2 = jnp.bitwise_or(t, b_lo)
        n1, n12, n2 = count3(c1, c12, c2)
        take1 = n1 >= _OMEGA
        cnt_lo = jnp.where(take1, n12, n2)
        take2 = cnt_lo >= _OMEGA
        t_new = jnp.where(take2,
                          jnp.where(take1, c12, c2),
                          jnp.where(take1, c1, t))
        cnt_new = jnp.where(take2, cnt_lo, jnp.where(take1, n1, cnt_t))
        return t_new, cnt_new

    def body(state):
        bit, t, cnt_t = state
        t, cnt_t = pair_step(bit, t, cnt_t)
        t, cnt_t = pair_step(bit - 2, t, cnt_t)
        return bit - 4, t, cnt_t

    _, t, _ = jax.lax.while_loop(cond, body, (jnp.int32(15), t0, cnt_t0))
    return jnp.where(a >= t, ct, 0.0)


def _star_kernel(x_ref, pw_ref, wd_ref, wm_ref, wa_ref, ba_ref, md_ref, z_ref,
                 pw_vmem, pw_sem):
    f32 = jnp.float32
    pw_copy = pltpu.make_async_copy(pw_ref, pw_vmem, pw_sem)
    pw_copy.start()
    wd = wd_ref[...]
    x = x_ref[...]

    bt = jax.lax.dot_general(0.5 * wd, x, (((1,), (1,)), ((), ())),
                             preferred_element_type=f32)
    g = jax.lax.dot_general(wd, wd, (((0,), (0,)), ((), ())),
                            preferred_element_type=f32)
    ii = jax.lax.broadcasted_iota(jnp.int32, (_TWO_W, _TWO_W), 0)
    jj = jax.lax.broadcasted_iota(jnp.int32, (_TWO_W, _TWO_W), 1)
    s_mat = jnp.where(ii == jj, f32(1.0), f32(0.0)) - 0.5 * g

    zt = _hard_thr_t(bt)

    def iter_body(_, zt):
        ct = bt + jax.lax.dot_general(s_mat, zt, (((1,), (0,)), ((), ())),
                                      preferred_element_type=f32)
        return _hard_thr_t(ct)

    zt = jax.lax.fori_loop(0, _N_ITERS, iter_body, zt)
    z = jnp.swapaxes(zt, 0, 1)
    z_ref[...] = z

    p = z[:, :_W] * z[:, :_W] + z[:, _W:] * z[:, _W:]
    md = jnp.sum(p, axis=0, keepdims=True)
    md_n = (md - jnp.min(md)) / (jnp.max(md) - jnp.min(md) + 1e-8)

    pw_copy.wait()
    rows = []
    for w in range(_NPV):
        h = pw_vmem[w * _BATCH:(w + 1) * _BATCH, :]
        ph = h[:, :_W] * h[:, :_W] + h[:, _W:] * h[:, _W:]
        rows.append(jnp.sum(ph, axis=0, keepdims=True))
    mdp = jnp.concatenate(rows, axis=0)
    lo = jnp.min(mdp, axis=1, keepdims=True)
    hi = jnp.max(mdp, axis=1, keepdims=True)
    mdp_n = (mdp - lo) / (hi - lo + 1e-8)

    att = jnp.sum(mdp_n * md_n, axis=1, keepdims=True)
    att = att / _SQRT_W
    e = jnp.exp(att - jnp.max(att))
    sm = e / jnp.sum(e)
    a = jnp.sum(mdp_n * sm, axis=0, keepdims=True)

    am = jax.nn.sigmoid(jax.lax.dot_general(a, wm_ref[...], (((1,), (1,)), ((), ())),
                                            preferred_element_type=f32))
    aa = jax.nn.relu(jax.lax.dot_general(a, wa_ref[...], (((1,), (1,)), ((), ())),
                                         preferred_element_type=f32) + ba_ref[...])
    mo = (md_n + aa) * am
    md_ref[...] = (mo - jnp.min(mo)) / (jnp.max(mo) - jnp.min(mo) + 1e-8)


def _star_call(x, pw, wd, wm, wa, ba, interpret=False):
    return pl.pallas_call(
        _star_kernel,
        in_specs=[
            pl.BlockSpec(memory_space=pltpu.MemorySpace.VMEM),
            pl.BlockSpec(memory_space=pltpu.MemorySpace.HBM),
            pl.BlockSpec(memory_space=pltpu.MemorySpace.VMEM),
            pl.BlockSpec(memory_space=pltpu.MemorySpace.VMEM),
            pl.BlockSpec(memory_space=pltpu.MemorySpace.VMEM),
            pl.BlockSpec(memory_space=pltpu.MemorySpace.VMEM),
        ],
        scratch_shapes=[
            pltpu.VMEM((_NPV * _BATCH, _TWO_W), jnp.float32),
            pltpu.SemaphoreType.DMA,
        ],
        out_shape=(
            jax.ShapeDtypeStruct((1, _W), jnp.float32),
            jax.ShapeDtypeStruct((_BATCH, _TWO_W), jnp.float32),
        ),
        interpret=interpret,
    )(x, pw, wd, wm, wa, ba)


def kernel(x, prev_windows, W_d, Wm, Wa, ba):
    pw = prev_windows.reshape(-1, prev_windows.shape[-1])
    md, z = _star_call(x, pw, W_d[0], Wm, Wa, ba.reshape(1, -1))
    return md.reshape(-1), z
